# baseline (speedup 1.0000x reference)
import functools

import jax
import jax.numpy as jnp
from jax import lax
from jax.experimental import pallas as pl
from jax.experimental.pallas import tpu as pltpu
from jax.experimental.pallas import tpu_sc as plsc

IN_DIM = 2048
HID = 1024
E = 8
NC = 2
GATE_H = 256
TOKENS = 4096
ASSIGN = 2 * TOKENS
BLK = 256
NBLK = ASSIGN // BLK + E
CAP = NBLK * BLK
PAD = CAP - ASSIGN
NW = 32


def _gelu(v):
    return v * 0.5 * (1.0 + lax.erf(v * 0.7071067811865476))


def _route_kernel(x_ref, Wg1_ref, bg1_ref, Wg2_ref, bg2_ref,
                  w_ref, pos_ref, pad_ref, be_ref):
    x = x_ref[...]
    g = _gelu(jnp.dot(x, Wg1_ref[...], preferred_element_type=jnp.float32)
              + bg1_ref[...])
    gl = jnp.dot(g, Wg2_ref[...], preferred_element_type=jnp.float32) \
        + bg2_ref[...]

    iota_e = lax.broadcasted_iota(jnp.int32, gl.shape, 1)
    m1 = jnp.max(gl, axis=-1, keepdims=True)
    i1 = jnp.min(jnp.where(gl == m1, iota_e, E), axis=-1, keepdims=True)
    oh1 = (iota_e == i1)
    masked = jnp.where(oh1, -jnp.inf, gl)
    m2 = jnp.max(masked, axis=-1, keepdims=True)
    i2 = jnp.min(jnp.where(masked == m2, iota_e, E), axis=-1, keepdims=True)
    oh2 = (iota_e == i2)
    e2 = jnp.exp(m2 - m1)
    w1 = 1.0 / (1.0 + e2)
    w2 = e2 * w1
    w_ref[...] = jnp.concatenate([w1, w2], axis=1)

    o1 = oh1.astype(jnp.float32)
    o2 = oh2.astype(jnp.float32)
    osum = o1 + o2
    inc = osum
    s = 1
    while s < TOKENS:
        inc = inc + jnp.concatenate(
            [jnp.zeros((s, E), jnp.float32), inc[:-s, :]], axis=0)
        s *= 2
    excl = inc - osum
    counts = inc[TOKENS - 1:TOKENS, :]

    padded = jnp.floor((counts + (BLK - 1)) / BLK) * BLK
    tri_e = (lax.broadcasted_iota(jnp.int32, (E, E), 0)
             < lax.broadcasted_iota(jnp.int32, (E, E), 1)).astype(jnp.float32)
    off = jnp.dot(padded, tri_e, preferred_element_type=jnp.float32)
    end = off + padded

    base = off + excl
    pos0 = jnp.sum(jnp.where(oh1, base, 0.0), axis=1, keepdims=True)
    pos1 = jnp.sum(jnp.where(oh2, base + o1, 0.0), axis=1, keepdims=True)
    pos_ref[...] = jnp.concatenate([pos0, pos1], axis=1).astype(jnp.int32)

    total = jnp.sum(padded, axis=1, keepdims=True)
    gsz = jnp.concatenate([padded - counts, CAP - total], axis=1)
    gstart = jnp.concatenate([off + counts, total], axis=1)
    tri_g = (lax.broadcasted_iota(jnp.int32, (E + 1, E + 1), 0)
             < lax.broadcasted_iota(jnp.int32, (E + 1, E + 1), 1)
             ).astype(jnp.float32)
    cumg = jnp.dot(gsz, tri_g, preferred_element_type=jnp.float32)
    i_pad = lax.broadcasted_iota(jnp.int32, (PAD, 1), 0).astype(jnp.float32)
    in_gap = jnp.logical_and(cumg <= i_pad, i_pad < cumg + gsz)
    pad_pos = jnp.sum(jnp.where(in_gap, gstart - cumg, 0.0), axis=1) \
        + i_pad[:, 0]
    pad_ref[...] = pad_pos.astype(jnp.int32)[None, :]

    jb = lax.broadcasted_iota(jnp.int32, (NBLK, 1), 0).astype(jnp.float32) * BLK
    be = jnp.sum((end <= jb).astype(jnp.int32), axis=1)
    be_ref[...] = jnp.minimum(be, E - 1)[None, :]


def _route(x, Wg1, bg1, Wg2, bg2):
    return pl.pallas_call(
        _route_kernel,
        out_shape=(
            jax.ShapeDtypeStruct((TOKENS, 2), jnp.float32),
            jax.ShapeDtypeStruct((TOKENS, 2), jnp.int32),
            jax.ShapeDtypeStruct((1, PAD), jnp.int32),
            jax.ShapeDtypeStruct((1, NBLK), jnp.int32),
        ),
    )(x, Wg1, bg1.reshape(1, GATE_H), Wg2, bg2.reshape(1, E))


_SC_MESH = dict(core_axis_name="c", subcore_axis_name="s")


def _sc_wid():
    return lax.axis_index("s") * 2 + lax.axis_index("c")


def _k2a_body(idx_hbm, out_hbm, iv128, vv128, iv64, vv64, sem):
    wid = _sc_wid()
    base = wid * (CAP // NW)
    iota16 = lax.iota(jnp.int32, 16)
    for ofs, n, iv, vv in ((0, 128, iv128, vv128),
                           (128, 128, iv128, vv128),
                           (256, 64, iv64, vv64)):
        pltpu.sync_copy(idx_hbm.at[pl.ds(base + ofs, n)], iv)
        for s in range(n // 16):
            a_vec = (base + ofs + s * 16) + iota16
            v = jnp.where(a_vec < ASSIGN,
                          lax.shift_right_logical(a_vec, 1), 0)
            vv[pl.ds(s * 16, 16)] = v
        pltpu.async_copy(vv, out_hbm.at[iv], sem).wait()


def _sc_scatter(idx_all):
    k = functools.partial(
        pl.kernel,
        mesh=plsc.VectorSubcoreMesh(**_SC_MESH),
        out_type=jax.ShapeDtypeStruct((CAP,), jnp.int32),
        scratch_types=[
            pltpu.VMEM((128,), jnp.int32),
            pltpu.VMEM((128,), jnp.int32),
            pltpu.VMEM((64,), jnp.int32),
            pltpu.VMEM((64,), jnp.int32),
            pltpu.SemaphoreType.DMA,
        ],
    )(_k2a_body)
    return k(idx_all)


_ROWS_PER_W = CAP // NW
_GCHUNK = 32


def _k2b_body(x_hbm, tok_hbm, xs_hbm, tokv, rows, sem):
    wid = _sc_wid()
    base = wid * _ROWS_PER_W
    for c in range(_ROWS_PER_W // _GCHUNK):
        start = base + c * _GCHUNK
        pltpu.sync_copy(tok_hbm.at[pl.ds(start, _GCHUNK)], tokv)
        pltpu.async_copy(x_hbm.at[tokv], rows, sem).wait()
        pltpu.sync_copy(rows, xs_hbm.at[pl.ds(start, _GCHUNK)])


def _sc_gather(x, sorted_tok):
    k = functools.partial(
        pl.kernel,
        mesh=plsc.VectorSubcoreMesh(**_SC_MESH),
        out_type=jax.ShapeDtypeStruct((CAP, IN_DIM), jnp.float32),
        scratch_types=[
            pltpu.VMEM((_GCHUNK,), jnp.int32),
            pltpu.VMEM((_GCHUNK, IN_DIM), jnp.float32),
            pltpu.SemaphoreType.DMA,
        ],
    )(_k2b_body)
    return k(x, sorted_tok)


def _mlp_kernel(be_ref, xs_ref, W1_ref, b1_ref, W2_ref, b2_ref,
                W3_ref, b3_ref, out_ref):
    h1 = _gelu(jnp.dot(xs_ref[...], W1_ref[0],
                       preferred_element_type=jnp.float32) + b1_ref[0])
    h2 = _gelu(jnp.dot(h1, W2_ref[0],
                       preferred_element_type=jnp.float32) + b2_ref[0])
    out_ref[...] = jnp.dot(h2, W3_ref[0],
                           preferred_element_type=jnp.float32) + b3_ref[0]


def _expert_mlp(be, xs, W1, b1, W2, b2, W3, b3):
    grid_spec = pltpu.PrefetchScalarGridSpec(
        num_scalar_prefetch=1,
        grid=(NBLK,),
        in_specs=[
            pl.BlockSpec((BLK, IN_DIM), lambda j, be: (j, 0)),
            pl.BlockSpec((1, IN_DIM, HID), lambda j, be: (be[j], 0, 0)),
            pl.BlockSpec((1, 1, HID), lambda j, be: (be[j], 0, 0)),
            pl.BlockSpec((1, HID, HID // 2), lambda j, be: (be[j], 0, 0)),
            pl.BlockSpec((1, 1, HID // 2), lambda j, be: (be[j], 0, 0)),
            pl.BlockSpec((1, HID // 2, NC), lambda j, be: (be[j], 0, 0)),
            pl.BlockSpec((1, 1, NC), lambda j, be: (be[j], 0, 0)),
        ],
        out_specs=pl.BlockSpec((BLK, NC), lambda j, be: (j, 0)),
    )
    return pl.pallas_call(
        _mlp_kernel,
        grid_spec=grid_spec,
        out_shape=jax.ShapeDtypeStruct((CAP, NC), jnp.float32),
    )(be, xs, W1, b1.reshape(E, 1, HID), W2, b2.reshape(E, 1, HID // 2),
      W3, b3.reshape(E, 1, NC))


_TOK_PER_W = TOKENS // NW


def _k4_body(o3_hbm, w0_hbm, w1_hbm, p0_hbm, p1_hbm, t_hbm, out_hbm,
             w0v, w1v, p0v, p1v, idxb, v00, v01, v10, v11, tv, ob, sem):
    wid = _sc_wid()
    tb = wid * _TOK_PER_W
    pltpu.sync_copy(w0_hbm.at[pl.ds(tb, _TOK_PER_W)], w0v)
    pltpu.sync_copy(w1_hbm.at[pl.ds(tb, _TOK_PER_W)], w1v)
    pltpu.sync_copy(p0_hbm.at[pl.ds(tb, _TOK_PER_W)], p0v)
    pltpu.sync_copy(p1_hbm.at[pl.ds(tb, _TOK_PER_W)], p1v)
    pltpu.sync_copy(t_hbm, tv)
    inv_t = 1.0 / jnp.maximum(tv[...], 1e-6)
    for pv, dsts in ((p0v, (v00, v01)), (p1v, (v10, v11))):
        for c, dst in enumerate(dsts):
            for g in range(_TOK_PER_W // 16):
                sl = pl.ds(g * 16, 16)
                idxb[sl] = pv[sl] * NC + c
            pltpu.async_copy(o3_hbm.at[idxb], dst, sem).wait()
    for c, (a, b) in enumerate(((v00, v10), (v01, v11))):
        for g in range(_TOK_PER_W // 16):
            sl = pl.ds(g * 16, 16)
            ob[sl] = (w0v[sl] * a[sl] + w1v[sl] * b[sl]) * inv_t
        pltpu.sync_copy(ob, out_hbm.at[pl.ds(c * TOKENS + tb, _TOK_PER_W)])


def _sc_combine(o3_flat, w0, w1, p0, p1, temp16):
    k = functools.partial(
        pl.kernel,
        mesh=plsc.VectorSubcoreMesh(**_SC_MESH),
        out_type=jax.ShapeDtypeStruct((TOKENS * NC,), jnp.float32),
        scratch_types=[
            pltpu.VMEM((_TOK_PER_W,), jnp.float32),
            pltpu.VMEM((_TOK_PER_W,), jnp.float32),
            pltpu.VMEM((_TOK_PER_W,), jnp.int32),
            pltpu.VMEM((_TOK_PER_W,), jnp.int32),
            pltpu.VMEM((_TOK_PER_W,), jnp.int32),
            pltpu.VMEM((_TOK_PER_W,), jnp.float32),
            pltpu.VMEM((_TOK_PER_W,), jnp.float32),
            pltpu.VMEM((_TOK_PER_W,), jnp.float32),
            pltpu.VMEM((_TOK_PER_W,), jnp.float32),
            pltpu.VMEM((16,), jnp.float32),
            pltpu.VMEM((_TOK_PER_W,), jnp.float32),
            pltpu.SemaphoreType.DMA,
        ],
    )(_k4_body)
    return k(o3_flat, w0, w1, p0, p1, temp16)


def kernel(x, W1, b1, W2, b2, W3, b3, Wg1, bg1, Wg2, bg2, temperature):
    w, pos, pad, be = _route(x, Wg1, bg1, Wg2, bg2)
    idx_all = jnp.concatenate([pos.reshape(ASSIGN), pad.reshape(PAD)])
    sorted_tok = _sc_scatter(idx_all)
    xs = _sc_gather(x, sorted_tok)
    o3 = _expert_mlp(be.reshape(NBLK), xs, W1, b1, W2, b2, W3, b3)
    temp16 = jnp.broadcast_to(temperature.reshape(1), (16,))
    out = _sc_combine(o3.reshape(CAP * NC), w[:, 0], w[:, 1],
                      pos[:, 0], pos[:, 1], temp16)
    return out.reshape(NC, TOKENS).T
